# baseline (speedup 1.0000x reference)
import functools

import jax
import jax.numpy as jnp
from jax import lax
from jax.experimental import pallas as pl
from jax.experimental.pallas import tpu as pltpu
from jax.experimental.pallas import tpu_sc as plsc

B = 16384
D = 64
NC = 2
NS = 16
NW = NC * NS
B_PER_W = B // NW
CH = 64
NCH = B_PER_W // CH


def _gather_body(idx_hbm, w2_hbm, out_hbm, idx_v, blk_v, blocks_v, out_v, gsem):
    wid = lax.axis_index("s") * NC + lax.axis_index("c")
    base = wid * B_PER_W
    pltpu.sync_copy(idx_hbm.at[wid], idx_v)

    iota = lax.iota(jnp.int32, 16)

    def blkcompute(i, carry):
        v = idx_v[pl.ds(i * 16, 16)]
        blk_v[pl.ds(i * 16, 16)] = lax.shift_right_logical(v, 1)
        return carry

    lax.fori_loop(0, B_PER_W // 16, blkcompute, 0)

    def fire(ch, slot):
        return pltpu.async_copy(
            w2_hbm.at[blk_v.at[pl.ds(ch * CH, CH)]],
            blocks_v.at[slot],
            gsem,
        )

    pending = fire(0, 0)
    for ch in range(NCH):
        cur = ch % 2
        pending.wait()
        if ch + 1 < NCH:
            pending = fire(ch + 1, 1 - cur)

        def extract(g, carry, ch=ch, cur=cur):
            slots16 = g * 16 + iota
            idxvec = idx_v[pl.ds(ch * CH + g * 16, 16)]
            colbase = lax.bitwise_and(idxvec, 1) * 64
            rows16 = ch * CH + slots16
            for c in range(D):
                vals = plsc.load_gather(
                    blocks_v.at[cur], [slots16, colbase + c]
                )
                plsc.store_scatter(
                    out_v, [rows16, jnp.full((16,), c, jnp.int32)], vals
                )
            return carry

        lax.fori_loop(0, CH // 16, extract, 0)

    pltpu.sync_copy(out_v, out_hbm.at[pl.ds(base, B_PER_W)])


@jax.jit
def kernel(h, weight):
    idx = h.reshape(NW, B_PER_W).astype(jnp.int32)
    w2 = weight.reshape(500000, 128)
    mesh = plsc.VectorSubcoreMesh(core_axis_name="c", subcore_axis_name="s")
    run = pl.kernel(
        _gather_body,
        out_type=jax.ShapeDtypeStruct((B, D), jnp.float32),
        mesh=mesh,
        scratch_types=[
            pltpu.VMEM((B_PER_W,), jnp.int32),
            pltpu.VMEM((B_PER_W,), jnp.int32),
            pltpu.VMEM((2, CH, 128), jnp.float32),
            pltpu.VMEM((B_PER_W, D), jnp.float32),
            pltpu.SemaphoreType.DMA,
        ],
        compiler_params=pltpu.CompilerParams(needs_layout_passes=False),
    )
    return run(idx, w2)


# speedup vs baseline: 1.7948x; 1.7948x over previous
import functools

import jax
import jax.numpy as jnp
from jax import lax
from jax.experimental import pallas as pl
from jax.experimental.pallas import tpu as pltpu
from jax.experimental.pallas import tpu_sc as plsc

B = 16384
D = 64
NC = 2
NS = 16
NW = NC * NS
B_PER_W = B // NW


def _gather_body(idx_hbm, table_hbm, out_hbm, idx_v, rows_v, gsem):
    wid = lax.axis_index("s") * NC + lax.axis_index("c")
    base = wid * B_PER_W
    pltpu.sync_copy(idx_hbm.at[wid], idx_v)

    def group(g, carry):
        vec = idx_v[pl.ds(g * 16, 16)]
        for j in range(16):
            pltpu.async_copy(
                table_hbm.at[pl.ds(vec[j], 1)],
                rows_v.at[pl.ds(g * 16 + j, 1)],
                gsem,
            )
        return carry

    lax.fori_loop(0, B_PER_W // 16, group, 0)
    pltpu.make_async_copy(table_hbm.at[pl.ds(0, B_PER_W)], rows_v, gsem).wait()
    pltpu.sync_copy(rows_v, out_hbm.at[pl.ds(base, B_PER_W)])


@jax.jit
def kernel(h, weight):
    idx = h.reshape(NW, B_PER_W).astype(jnp.int32)
    mesh = plsc.VectorSubcoreMesh(core_axis_name="c", subcore_axis_name="s")
    run = pl.kernel(
        _gather_body,
        out_type=jax.ShapeDtypeStruct((B, D), jnp.float32),
        mesh=mesh,
        scratch_types=[
            pltpu.VMEM((B_PER_W,), jnp.int32),
            pltpu.VMEM((B_PER_W, D), jnp.float32),
            pltpu.SemaphoreType.DMA,
        ],
    )
    return run(idx, weight)
